# baseline (speedup 1.0000x reference)
import jax
import jax.numpy as jnp
from jax.experimental import pallas as pl

BN = 1024


def _vq_block(x_ref, w_ref, x2_ref, w2_ref, out_ref):
    x = x_ref[:, :]
    w = w_ref[:, :]
    xw = jax.lax.dot_general(
        x, w, (((1,), (1,)), ((), ())),
        preferred_element_type=jnp.float32)
    dists = x2_ref[:, :] - 2.0 * xw + w2_ref[:, :]
    kk = dists.shape[1]
    mn = jnp.min(dists, axis=1, keepdims=True)
    col = jax.lax.broadcasted_iota(jnp.int32, dists.shape, 1)
    out_ref[:] = jnp.min(jnp.where(dists == mn, col, kk), axis=1)


def kernel(z_e_x, W):
    lead = z_e_x.shape[:-1]
    D = z_e_x.shape[-1]
    flat = z_e_x.reshape(-1, D)
    N = flat.shape[0]
    K = W.shape[0]
    x2 = jnp.sum(flat * flat, axis=1, keepdims=True)
    w2 = jnp.sum(W * W, axis=1)[None, :]
    grid = (N // BN,)
    idx = pl.pallas_call(
        _vq_block,
        grid=grid,
        in_specs=[
            pl.BlockSpec((BN, D), lambda i: (i, 0)),
            pl.BlockSpec((K, D), lambda i: (0, 0)),
            pl.BlockSpec((BN, 1), lambda i: (i, 0)),
            pl.BlockSpec((1, K), lambda i: (0, 0)),
        ],
        out_specs=pl.BlockSpec((BN,), lambda i: (i,)),
        out_shape=jax.ShapeDtypeStruct((N,), jnp.int32),
    )(flat, W, x2, w2)
    return idx.reshape(lead).astype(jnp.int64)


# speedup vs baseline: 1.6607x; 1.6607x over previous
import jax
import jax.numpy as jnp
from jax.experimental import pallas as pl

BN = 1024


def _vq_block(x_ref, w_ref, x2_ref, w2_ref, out_ref):
    x = x_ref[:, :]
    w = w_ref[:, :]
    K = w.shape[0]
    x2 = x + x
    m = jax.lax.dot_general(
        w, x2, (((1,), (1,)), ((), ())),
        preferred_element_type=jnp.float32)
    dists = x2_ref[:, :] - m + w2_ref[:, :]
    mn = jnp.min(dists, axis=0, keepdims=True)
    row = jax.lax.broadcasted_iota(jnp.int32, dists.shape, 0)
    out_ref[0, 0, :] = jnp.min(jnp.where(dists == mn, row, K), axis=0)


def kernel(z_e_x, W):
    lead = z_e_x.shape[:-1]
    D = z_e_x.shape[-1]
    flat = z_e_x.reshape(-1, D)
    N = flat.shape[0]
    K = W.shape[0]
    G = N // BN
    x2 = jnp.sum(flat * flat, axis=1)[None, :]
    w2 = jnp.sum(W * W, axis=1)[:, None]
    idx = pl.pallas_call(
        _vq_block,
        grid=(G,),
        in_specs=[
            pl.BlockSpec((BN, D), lambda i: (i, 0)),
            pl.BlockSpec((K, D), lambda i: (0, 0)),
            pl.BlockSpec((1, BN), lambda i: (0, i)),
            pl.BlockSpec((K, 1), lambda i: (0, 0)),
        ],
        out_specs=pl.BlockSpec((1, 1, BN), lambda i: (i, 0, 0)),
        out_shape=jax.ShapeDtypeStruct((G, 1, BN), jnp.int32),
    )(flat, W, x2, w2)
    return idx.reshape(lead).astype(jnp.int64)


# speedup vs baseline: 1.7951x; 1.0809x over previous
import jax
import jax.numpy as jnp
from jax.experimental import pallas as pl

BN = 2048


def _vq_block(x_ref, w_ref, x2_ref, w2_ref, rowf_ref, out_ref):
    x = x_ref[:, :]
    w = w_ref[:, :]
    K = w.shape[0]
    x2 = x + x
    m = jax.lax.dot_general(
        w, x2, (((1,), (1,)), ((), ())),
        preferred_element_type=jnp.float32)
    dists = x2_ref[:, :] - m + w2_ref[:, :]
    mn = jnp.min(dists, axis=0, keepdims=True)
    masked = jnp.where(dists == mn, rowf_ref[:, :], float(K))
    out_ref[0, 0, :] = jnp.min(masked, axis=0).astype(jnp.int32)


def kernel(z_e_x, W):
    lead = z_e_x.shape[:-1]
    D = z_e_x.shape[-1]
    flat = z_e_x.reshape(-1, D)
    N = flat.shape[0]
    K = W.shape[0]
    G = N // BN
    x2 = jnp.sum(flat * flat, axis=1)[None, :]
    w2 = jnp.sum(W * W, axis=1)[:, None]
    rowf = jnp.arange(K, dtype=jnp.float32)[:, None]
    idx = pl.pallas_call(
        _vq_block,
        grid=(G,),
        in_specs=[
            pl.BlockSpec((BN, D), lambda i: (i, 0)),
            pl.BlockSpec((K, D), lambda i: (0, 0)),
            pl.BlockSpec((1, BN), lambda i: (0, i)),
            pl.BlockSpec((K, 1), lambda i: (0, 0)),
            pl.BlockSpec((K, 1), lambda i: (0, 0)),
        ],
        out_specs=pl.BlockSpec((1, 1, BN), lambda i: (i, 0, 0)),
        out_shape=jax.ShapeDtypeStruct((G, 1, BN), jnp.int32),
    )(flat, W, x2, w2, rowf)
    return idx.reshape(lead).astype(jnp.int64)


# speedup vs baseline: 1.8587x; 1.0354x over previous
import jax
import jax.numpy as jnp
import numpy as np
from jax.experimental import pallas as pl

B_LEAD = 4


def _vq_block(x_ref, w_ref, x2_ref, w2_ref, rowf_ref, out_ref):
    bn = x_ref.shape[0] * x_ref.shape[1]
    d = x_ref.shape[2]
    x = x_ref[:, :, :].reshape(bn, d)
    w = w_ref[:, :]
    K = w.shape[0]
    x2 = x + x
    m = jax.lax.dot_general(
        w, x2, (((1,), (1,)), ((), ())),
        preferred_element_type=jnp.float32)
    dists = x2_ref[:, :] - m + w2_ref[:, :]
    mn = jnp.min(dists, axis=0, keepdims=True)
    masked = jnp.where(dists == mn, rowf_ref[:, :], float(K))
    out_ref[0, 0, :] = jnp.min(masked, axis=0).astype(jnp.int32)


def kernel(z_e_x, W):
    lead = z_e_x.shape[:-1]
    B, S, D = z_e_x.shape
    N = B * S
    K = W.shape[0]
    G = B // B_LEAD
    BN = B_LEAD * S
    x2 = jnp.sum(z_e_x * z_e_x, axis=-1).reshape(1, N)
    w2 = jnp.sum(W * W, axis=1)[:, None]
    rowf = jnp.asarray(np.arange(K, dtype=np.float32)[:, None])
    idx = pl.pallas_call(
        _vq_block,
        grid=(G,),
        in_specs=[
            pl.BlockSpec((B_LEAD, S, D), lambda i: (i, 0, 0)),
            pl.BlockSpec((K, D), lambda i: (0, 0)),
            pl.BlockSpec((1, BN), lambda i: (0, i)),
            pl.BlockSpec((K, 1), lambda i: (0, 0)),
            pl.BlockSpec((K, 1), lambda i: (0, 0)),
        ],
        out_specs=pl.BlockSpec((1, 1, BN), lambda i: (i, 0, 0)),
        out_shape=jax.ShapeDtypeStruct((G, 1, BN), jnp.int32),
    )(z_e_x, W, x2, w2, rowf)
    return idx.reshape(lead).astype(jnp.int64)


# speedup vs baseline: 2.0293x; 1.0918x over previous
import jax
import jax.numpy as jnp
import numpy as np
from jax.experimental import pallas as pl

B_LEAD = 4
CH = 64


def _vq_block(x_ref, w_ref, x2_ref, w2_ref, rowf_ref, out_ref):
    bn = x_ref.shape[0] * x_ref.shape[1]
    d = x_ref.shape[2]
    x = x_ref[:, :, :].reshape(bn, d)
    w = w_ref[:, :]
    K = w.shape[0]
    x2 = x2_ref[:, :]
    m = jax.lax.dot_general(
        w, x + x, (((1,), (1,)), ((), ())),
        preferred_element_type=jnp.float32)
    cur = (x2 - m[0:CH, :]) + w2_ref[0:CH, :]
    idx = jnp.broadcast_to(rowf_ref[0:CH, :], (CH, bn))
    for c in range(1, K // CH):
        lo = c * CH
        dc = (x2 - m[lo:lo + CH, :]) + w2_ref[lo:lo + CH, :]
        better = dc < cur
        idx = jnp.where(better, rowf_ref[lo:lo + CH, :], idx)
        cur = jnp.minimum(cur, dc)
    mn = jnp.min(cur, axis=0, keepdims=True)
    masked = jnp.where(cur == mn, idx, float(K))
    out_ref[0, 0, :] = jnp.min(masked, axis=0).astype(jnp.int32)


def kernel(z_e_x, W):
    lead = z_e_x.shape[:-1]
    B, S, D = z_e_x.shape
    N = B * S
    K = W.shape[0]
    G = B // B_LEAD
    BN = B_LEAD * S
    x2 = jnp.sum(z_e_x * z_e_x, axis=-1).reshape(1, N)
    w2 = jnp.sum(W * W, axis=1)[:, None]
    rowf = jnp.asarray(np.arange(K, dtype=np.float32)[:, None])
    idx = pl.pallas_call(
        _vq_block,
        grid=(G,),
        in_specs=[
            pl.BlockSpec((B_LEAD, S, D), lambda i: (i, 0, 0)),
            pl.BlockSpec((K, D), lambda i: (0, 0)),
            pl.BlockSpec((1, BN), lambda i: (0, i)),
            pl.BlockSpec((K, 1), lambda i: (0, 0)),
            pl.BlockSpec((K, 1), lambda i: (0, 0)),
        ],
        out_specs=pl.BlockSpec((1, 1, BN), lambda i: (i, 0, 0)),
        out_shape=jax.ShapeDtypeStruct((G, 1, BN), jnp.int32),
    )(z_e_x, W, x2, w2, rowf)
    return idx.reshape(lead).astype(jnp.int64)
